# baseline (speedup 1.0000x reference)
import functools

import jax
import jax.numpy as jnp
import numpy as np
from jax import lax
from jax.experimental import pallas as pl
from jax.experimental.pallas import tpu as pltpu
from jax.experimental.pallas import tpu_sc as plsc

_PERM = (7, 4, 2, 5, 3, 6, 0, 1)

_NW = 32


def kernel(x):
    n, s, d = x.shape
    rows = n // _NW
    mesh = plsc.VectorSubcoreMesh(core_axis_name="c", subcore_axis_name="s")

    @functools.partial(
        pl.kernel,
        mesh=mesh,
        out_type=jax.ShapeDtypeStruct((n, s, d), x.dtype),
    )
    def run(x_hbm, o_hbm):
        wid = lax.axis_index("s") * 2 + lax.axis_index("c")
        base = wid * rows
        for j, p in enumerate(_PERM):
            pltpu.sync_copy(
                x_hbm.at[pl.ds(base, rows), pl.ds(p, 1), :],
                o_hbm.at[pl.ds(base, rows), pl.ds(j, 1), :],
            )

    return run(x)


# speedup vs baseline: 31.1418x; 31.1418x over previous
import functools

import jax
import jax.numpy as jnp
import numpy as np
from jax import lax
from jax.experimental import pallas as pl
from jax.experimental.pallas import tpu as pltpu
from jax.experimental.pallas import tpu_sc as plsc

_PERM = (7, 4, 2, 5, 3, 6, 0, 1)

_NW = 32
_C = 16


def kernel(x):
    n, s, d = x.shape
    rows = n // _NW
    nchunks = rows // _C
    mesh = plsc.VectorSubcoreMesh(core_axis_name="c", subcore_axis_name="s")

    @functools.partial(
        pl.kernel,
        mesh=mesh,
        out_type=jax.ShapeDtypeStruct((n, s, d), x.dtype),
        scratch_types=[
            pltpu.VMEM((_C, s, d), jnp.float32),
            pltpu.VMEM((_C, s, d), jnp.float32),
            pltpu.SemaphoreType.DMA,
            pltpu.SemaphoreType.DMA,
        ],
    )
    def run(x_hbm, o_hbm, buf0, buf1, rsem, wsem):
        wid = lax.axis_index("s") * 2 + lax.axis_index("c")
        base = wid * rows
        bufs = (buf0, buf1)
        rcur = pltpu.async_copy(x_hbm.at[pl.ds(base, _C)], bufs[0], rsem)
        for i in range(nchunks):
            rb = base + i * _C
            rnext = None
            if i + 1 < nchunks:
                rnext = pltpu.async_copy(
                    x_hbm.at[pl.ds(base + (i + 1) * _C, _C)],
                    bufs[(i + 1) % 2], rsem)
            rcur.wait()
            handles = [
                pltpu.async_copy(
                    bufs[i % 2].at[:, pl.ds(p, 1), :],
                    o_hbm.at[pl.ds(rb, _C), pl.ds(j, 1), :],
                    wsem)
                for j, p in enumerate(_PERM)
            ]
            for h in handles:
                h.wait()
            rcur = rnext

    return run(x)


# speedup vs baseline: 31.5226x; 1.0122x over previous
import functools

import jax
import jax.numpy as jnp
import numpy as np
from jax import lax
from jax.experimental import pallas as pl
from jax.experimental.pallas import tpu as pltpu
from jax.experimental.pallas import tpu_sc as plsc

_PERM = (7, 4, 2, 5, 3, 6, 0, 1)

_NW = 32
_C = 16


def kernel(x):
    n, s, d = x.shape
    rows = n // _NW
    nchunks = rows // _C
    mesh = plsc.VectorSubcoreMesh(core_axis_name="c", subcore_axis_name="s")

    @functools.partial(
        pl.kernel,
        mesh=mesh,
        out_type=jax.ShapeDtypeStruct((n, s, d), x.dtype),
        scratch_types=[
            pltpu.VMEM((_C, s, d), jnp.float32),
            pltpu.VMEM((_C, s, d), jnp.float32),
            pltpu.SemaphoreType.DMA,
            pltpu.SemaphoreType.DMA,
        ],
    )
    def run(x_hbm, o_hbm, buf0, buf1, rsem, wsem):
        wid = lax.axis_index("s") * 2 + lax.axis_index("c")
        base = wid * rows
        bufs = (buf0, buf1)
        def read_chunk(i):
            rb = base + i * _C
            return [
                pltpu.async_copy(
                    x_hbm.at[pl.ds(rb, _C), pl.ds(p, 1), :],
                    bufs[i % 2].at[:, pl.ds(j, 1), :],
                    rsem)
                for j, p in enumerate(_PERM)
            ]

        rcur = read_chunk(0)
        for i in range(nchunks):
            rnext = read_chunk(i + 1) if i + 1 < nchunks else None
            for h in rcur:
                h.wait()
            w = pltpu.async_copy(
                bufs[i % 2], o_hbm.at[pl.ds(base + i * _C, _C)], wsem)
            w.wait()
            rcur = rnext

    return run(x)


# speedup vs baseline: 31.5875x; 1.0021x over previous
import functools

import jax
import jax.numpy as jnp
import numpy as np
from jax import lax
from jax.experimental import pallas as pl
from jax.experimental.pallas import tpu as pltpu
from jax.experimental.pallas import tpu_sc as plsc

_PERM = (7, 4, 2, 5, 3, 6, 0, 1)

_NW = 32
_C = 16


def kernel(x):
    n, s, d = x.shape
    rows = n // _NW
    nchunks = rows // _C
    mesh = plsc.VectorSubcoreMesh(core_axis_name="c", subcore_axis_name="s")

    @functools.partial(
        pl.kernel,
        mesh=mesh,
        out_type=jax.ShapeDtypeStruct((n, s, d), x.dtype),
        scratch_types=[
            pltpu.VMEM((_C, s, d), jnp.float32),
            pltpu.VMEM((_C, s, d), jnp.float32),
            pltpu.SemaphoreType.DMA,
            pltpu.SemaphoreType.DMA,
        ],
    )
    def run(x_hbm, o_hbm, buf0, buf1, rsem, wsem):
        wid = lax.axis_index("s") * 2 + lax.axis_index("c")
        base = wid * rows
        bufs = (buf0, buf1)
        def read_chunk(i):
            rb = base + i * _C
            return [
                pltpu.async_copy(
                    x_hbm.at[pl.ds(rb, _C), pl.ds(p, 1), :],
                    bufs[i % 2].at[:, pl.ds(j, 1), :],
                    rsem)
                for j, p in enumerate(_PERM)
            ]

        rcur = read_chunk(0)
        wh = [None, None]
        for i in range(nchunks):
            rnext = None
            if i + 1 < nchunks:
                if wh[(i + 1) % 2] is not None:
                    wh[(i + 1) % 2].wait()
                rnext = read_chunk(i + 1)
            for h in rcur:
                h.wait()
            wh[i % 2] = pltpu.async_copy(
                bufs[i % 2], o_hbm.at[pl.ds(base + i * _C, _C)], wsem)
            rcur = rnext
        for h in wh:
            if h is not None:
                h.wait()

    return run(x)
